# baseline (speedup 1.0000x reference)
import jax
import jax.numpy as jnp
from jax.experimental import pallas as pl
from jax.experimental.pallas import tpu as pltpu

NUM_TOKENS = 32768
HIDDEN = 768
NUM_EXPERTS = 8
LANES = 128
SUB = 512
NCHUNK = NUM_TOKENS // SUB
NBUF = 12


def _router_kernel(hid_ref, wt_ref, logits_ref, sel_ref, wgt_ref, var_ref,
                   ent_ref, *scratch):
    bufs = scratch[:NBUF]
    load_acc, ent_acc = scratch[NBUF], scratch[NBUF + 1]
    sems = scratch[NBUF + 2:]

    def copy_in(chunk):
        slot = chunk % NBUF
        pltpu.make_async_copy(
            hid_ref.at[pl.ds(chunk * SUB, SUB), :],
            bufs[slot],
            sems[slot],
        ).start()

    for c in range(NBUF):
        copy_in(c)

    wt = wt_ref[...]
    load_acc[...] = jnp.zeros((NUM_EXPERTS, 1), jnp.float32)
    ent_acc[...] = jnp.zeros((1, 1), jnp.float32)

    for c in range(NCHUNK):
        slot = c % NBUF
        pltpu.make_async_copy(
            hid_ref.at[pl.ds(c * SUB, SUB), :],
            bufs[slot],
            sems[slot],
        ).wait()
        x = bufs[slot][...]
        lt = jax.lax.dot_general(
            wt, x, (((0,), (1,)), ((), ())),
            preferred_element_type=jnp.float32)
        if c + NBUF < NCHUNK:
            copy_in(c + NBUF)

        l8 = lt[:NUM_EXPERTS, :]

        m = jnp.max(l8, axis=0, keepdims=True)
        e = jnp.exp(l8 - m)
        s = jnp.sum(e, axis=0, keepdims=True)
        probs = e / s

        sel = jnp.zeros((1, SUB), jnp.int32)
        for k in range(NUM_EXPERTS - 1, -1, -1):
            sel = jnp.where(l8[k:k + 1, :] == m, k, sel)

        cols = pl.ds(c * SUB, SUB)
        logits_ref[:, cols] = l8
        sel_ref[:, cols] = sel
        wgt_ref[:, cols] = 1.0 / s

        ent_tok = -jnp.sum(probs * jnp.log(probs + 1e-8), axis=0,
                           keepdims=True)
        ent_acc[...] += jnp.sum(ent_tok).reshape(1, 1)
        load_acc[...] += jnp.sum(probs, axis=1, keepdims=True)

    load = load_acc[...] / NUM_TOKENS
    mean = jnp.sum(load) / NUM_EXPERTS
    var = jnp.sum((load - mean) ** 2) / NUM_EXPERTS
    var_ref[...] = var.reshape(1, 1)
    ent_ref[...] = ent_acc[...] / NUM_TOKENS


@jax.jit
def kernel(hidden_states, W):
    wt = jnp.pad(W.T, ((0, 0), (0, LANES - NUM_EXPERTS)))

    out_types = (
        jax.ShapeDtypeStruct((NUM_EXPERTS, NUM_TOKENS), jnp.float32),
        jax.ShapeDtypeStruct((1, NUM_TOKENS), jnp.int32),
        jax.ShapeDtypeStruct((1, NUM_TOKENS), jnp.float32),
        jax.ShapeDtypeStruct((1, 1), jnp.float32),
        jax.ShapeDtypeStruct((1, 1), jnp.float32),
    )
    logits_t, sel_t, wgt_t, var, ent = pl.pallas_call(
        _router_kernel,
        in_specs=[
            pl.BlockSpec(memory_space=pltpu.MemorySpace.HBM),
            pl.BlockSpec(memory_space=pltpu.MemorySpace.VMEM),
        ],
        out_specs=(
            pl.BlockSpec(memory_space=pltpu.MemorySpace.VMEM),
            pl.BlockSpec(memory_space=pltpu.MemorySpace.VMEM),
            pl.BlockSpec(memory_space=pltpu.MemorySpace.VMEM),
            pl.BlockSpec(memory_space=pltpu.MemorySpace.VMEM),
            pl.BlockSpec(memory_space=pltpu.MemorySpace.VMEM),
        ),
        out_shape=out_types,
        scratch_shapes=(
            [pltpu.VMEM((SUB, HIDDEN), jnp.float32) for _ in range(NBUF)]
            + [pltpu.VMEM((NUM_EXPERTS, 1), jnp.float32),
               pltpu.VMEM((1, 1), jnp.float32)]
            + [pltpu.SemaphoreType.DMA for _ in range(NBUF)]
        ),
    )(hidden_states, wt)

    return (logits_t.T, sel_t.reshape(NUM_TOKENS, 1),
            wgt_t.reshape(NUM_TOKENS, 1), var.reshape(()), ent.reshape(()))


# speedup vs baseline: 1.0829x; 1.0829x over previous
import jax
import jax.numpy as jnp
from jax.experimental import pallas as pl
from jax.experimental.pallas import tpu as pltpu

NUM_TOKENS = 32768
HIDDEN = 768
NUM_EXPERTS = 8
LANES = 128
SUB = 2048
NCHUNK = NUM_TOKENS // SUB
NBUF = 6


def _router_kernel(hid_ref, wt_ref, logits_ref, sel_ref, wgt_ref, var_ref,
                   ent_ref, *scratch):
    bufs = scratch[:NBUF]
    load_acc, ent_acc = scratch[NBUF], scratch[NBUF + 1]
    sems = scratch[NBUF + 2:]

    def copy_in(chunk):
        slot = chunk % NBUF
        pltpu.make_async_copy(
            hid_ref.at[pl.ds(chunk * SUB, SUB), :],
            bufs[slot],
            sems[slot],
        ).start()

    for c in range(NBUF):
        copy_in(c)

    wt = wt_ref[...]
    load_acc[...] = jnp.zeros((NUM_EXPERTS, 1), jnp.float32)
    ent_acc[...] = jnp.zeros((1, 1), jnp.float32)

    for c in range(NCHUNK):
        slot = c % NBUF
        pltpu.make_async_copy(
            hid_ref.at[pl.ds(c * SUB, SUB), :],
            bufs[slot],
            sems[slot],
        ).wait()
        x = bufs[slot][...]
        lt = jax.lax.dot_general(
            wt, x, (((0,), (1,)), ((), ())),
            preferred_element_type=jnp.float32)
        if c + NBUF < NCHUNK:
            copy_in(c + NBUF)

        l8 = lt[:NUM_EXPERTS, :]

        m = jnp.max(l8, axis=0, keepdims=True)
        e = jnp.exp(l8 - m)
        s = jnp.sum(e, axis=0, keepdims=True)
        probs = e / s

        sel = jnp.zeros((1, SUB), jnp.int32)
        for k in range(NUM_EXPERTS - 1, -1, -1):
            sel = jnp.where(l8[k:k + 1, :] == m, k, sel)

        cols = pl.ds(c * SUB, SUB)
        logits_ref[:, cols] = l8
        sel_ref[:, cols] = sel
        wgt_ref[:, cols] = 1.0 / s

        ent_tok = -jnp.sum(probs * jnp.log(probs + 1e-8), axis=0,
                           keepdims=True)
        ent_acc[...] += jnp.sum(ent_tok).reshape(1, 1)
        load_acc[...] += jnp.sum(probs, axis=1, keepdims=True)

    load = load_acc[...] / NUM_TOKENS
    mean = jnp.sum(load) / NUM_EXPERTS
    var = jnp.sum((load - mean) ** 2) / NUM_EXPERTS
    var_ref[...] = var.reshape(1, 1)
    ent_ref[...] = ent_acc[...] / NUM_TOKENS


@jax.jit
def kernel(hidden_states, W):
    wt = jnp.pad(W.T, ((0, 0), (0, LANES - NUM_EXPERTS)))

    out_types = (
        jax.ShapeDtypeStruct((NUM_EXPERTS, NUM_TOKENS), jnp.float32),
        jax.ShapeDtypeStruct((1, NUM_TOKENS), jnp.int32),
        jax.ShapeDtypeStruct((1, NUM_TOKENS), jnp.float32),
        jax.ShapeDtypeStruct((1, 1), jnp.float32),
        jax.ShapeDtypeStruct((1, 1), jnp.float32),
    )
    logits_t, sel_t, wgt_t, var, ent = pl.pallas_call(
        _router_kernel,
        in_specs=[
            pl.BlockSpec(memory_space=pltpu.MemorySpace.HBM),
            pl.BlockSpec(memory_space=pltpu.MemorySpace.VMEM),
        ],
        out_specs=(
            pl.BlockSpec(memory_space=pltpu.MemorySpace.VMEM),
            pl.BlockSpec(memory_space=pltpu.MemorySpace.VMEM),
            pl.BlockSpec(memory_space=pltpu.MemorySpace.VMEM),
            pl.BlockSpec(memory_space=pltpu.MemorySpace.VMEM),
            pl.BlockSpec(memory_space=pltpu.MemorySpace.VMEM),
        ),
        out_shape=out_types,
        scratch_shapes=(
            [pltpu.VMEM((SUB, HIDDEN), jnp.float32) for _ in range(NBUF)]
            + [pltpu.VMEM((NUM_EXPERTS, 1), jnp.float32),
               pltpu.VMEM((1, 1), jnp.float32)]
            + [pltpu.SemaphoreType.DMA for _ in range(NBUF)]
        ),
    )(hidden_states, wt)

    return (logits_t.T, sel_t.reshape(NUM_TOKENS, 1),
            wgt_t.reshape(NUM_TOKENS, 1), var.reshape(()), ent.reshape(()))


# speedup vs baseline: 1.0919x; 1.0083x over previous
import jax
import jax.numpy as jnp
from jax.experimental import pallas as pl
from jax.experimental.pallas import tpu as pltpu

NUM_TOKENS = 32768
HIDDEN = 768
NUM_EXPERTS = 8
LANES = 128
SUB = 1024
NCHUNK = NUM_TOKENS // SUB
NBUF = 8


def _router_kernel(hid_ref, wt_ref, logits_ref, sel_ref, wgt_ref, var_ref,
                   ent_ref, *scratch):
    bufs = scratch[:NBUF]
    load_acc, ent_acc = scratch[NBUF], scratch[NBUF + 1]
    sems = scratch[NBUF + 2:]

    def copy_in(chunk):
        slot = chunk % NBUF
        pltpu.make_async_copy(
            hid_ref.at[pl.ds(chunk * SUB, SUB // 2), :],
            bufs[slot].at[: SUB // 2],
            sems[2 * slot],
        ).start()
        pltpu.make_async_copy(
            hid_ref.at[pl.ds(chunk * SUB + SUB // 2, SUB // 2), :],
            bufs[slot].at[SUB // 2:],
            sems[2 * slot + 1],
        ).start()

    for c in range(NBUF):
        copy_in(c)

    wt = wt_ref[...]
    load_acc[...] = jnp.zeros((NUM_EXPERTS, 1), jnp.float32)
    ent_acc[...] = jnp.zeros((1, 1), jnp.float32)

    for c in range(NCHUNK):
        slot = c % NBUF
        pltpu.make_async_copy(
            hid_ref.at[pl.ds(c * SUB, SUB // 2), :],
            bufs[slot].at[: SUB // 2],
            sems[2 * slot],
        ).wait()
        pltpu.make_async_copy(
            hid_ref.at[pl.ds(c * SUB + SUB // 2, SUB // 2), :],
            bufs[slot].at[SUB // 2:],
            sems[2 * slot + 1],
        ).wait()
        x = bufs[slot][...]
        lt = jax.lax.dot_general(
            wt, x, (((0,), (1,)), ((), ())),
            preferred_element_type=jnp.float32)
        if c + NBUF < NCHUNK:
            copy_in(c + NBUF)

        l8 = lt[:NUM_EXPERTS, :]

        m = jnp.max(l8, axis=0, keepdims=True)
        e = jnp.exp(l8 - m)
        s = jnp.sum(e, axis=0, keepdims=True)
        probs = e / s

        sel = jnp.zeros((1, SUB), jnp.int32)
        for k in range(NUM_EXPERTS - 1, -1, -1):
            sel = jnp.where(l8[k:k + 1, :] == m, k, sel)

        cols = pl.ds(c * SUB, SUB)
        logits_ref[:, cols] = l8
        sel_ref[:, cols] = sel
        wgt_ref[:, cols] = 1.0 / s

        ent_tok = -jnp.sum(probs * jnp.log(probs + 1e-8), axis=0,
                           keepdims=True)
        ent_acc[...] += jnp.sum(ent_tok).reshape(1, 1)
        load_acc[...] += jnp.sum(probs, axis=1, keepdims=True)

    load = load_acc[...] / NUM_TOKENS
    mean = jnp.sum(load) / NUM_EXPERTS
    var = jnp.sum((load - mean) ** 2) / NUM_EXPERTS
    var_ref[...] = var.reshape(1, 1)
    ent_ref[...] = ent_acc[...] / NUM_TOKENS


@jax.jit
def kernel(hidden_states, W):
    wt = jnp.pad(W.T, ((0, 0), (0, LANES - NUM_EXPERTS)))

    out_types = (
        jax.ShapeDtypeStruct((NUM_EXPERTS, NUM_TOKENS), jnp.float32),
        jax.ShapeDtypeStruct((1, NUM_TOKENS), jnp.int32),
        jax.ShapeDtypeStruct((1, NUM_TOKENS), jnp.float32),
        jax.ShapeDtypeStruct((1, 1), jnp.float32),
        jax.ShapeDtypeStruct((1, 1), jnp.float32),
    )
    logits_t, sel_t, wgt_t, var, ent = pl.pallas_call(
        _router_kernel,
        in_specs=[
            pl.BlockSpec(memory_space=pltpu.MemorySpace.HBM),
            pl.BlockSpec(memory_space=pltpu.MemorySpace.VMEM),
        ],
        out_specs=(
            pl.BlockSpec(memory_space=pltpu.MemorySpace.VMEM),
            pl.BlockSpec(memory_space=pltpu.MemorySpace.VMEM),
            pl.BlockSpec(memory_space=pltpu.MemorySpace.VMEM),
            pl.BlockSpec(memory_space=pltpu.MemorySpace.VMEM),
            pl.BlockSpec(memory_space=pltpu.MemorySpace.VMEM),
        ),
        out_shape=out_types,
        scratch_shapes=(
            [pltpu.VMEM((SUB, HIDDEN), jnp.float32) for _ in range(NBUF)]
            + [pltpu.VMEM((NUM_EXPERTS, 1), jnp.float32),
               pltpu.VMEM((1, 1), jnp.float32)]
            + [pltpu.SemaphoreType.DMA for _ in range(2 * NBUF)]
        ),
    )(hidden_states, wt)

    return (logits_t.T, sel_t.reshape(NUM_TOKENS, 1),
            wgt_t.reshape(NUM_TOKENS, 1), var.reshape(()), ent.reshape(()))


# speedup vs baseline: 1.1026x; 1.0098x over previous
import jax
import jax.numpy as jnp
from jax.experimental import pallas as pl
from jax.experimental.pallas import tpu as pltpu

NUM_TOKENS = 32768
HIDDEN = 768
NUM_EXPERTS = 8
LANES = 128
SUB = 1024
NCHUNK = NUM_TOKENS // SUB
NBUF = 8


def _router_kernel(hid_ref, wt_ref, logits_ref, sel_ref, wgt_ref, var_ref,
                   ent_ref, *scratch):
    bufs = scratch[:NBUF]
    load_acc, ent_acc = scratch[NBUF], scratch[NBUF + 1]
    sems = scratch[NBUF + 2:]

    def copy_in(chunk):
        slot = chunk % NBUF
        pltpu.make_async_copy(
            hid_ref.at[pl.ds(chunk * SUB, SUB), :],
            bufs[slot],
            sems[slot],
        ).start()

    for c in range(NBUF):
        copy_in(c)

    wt = wt_ref[...]
    load_acc[...] = jnp.zeros((NUM_EXPERTS, 1), jnp.float32)
    ent_acc[...] = jnp.zeros((1, 1), jnp.float32)

    for c in range(NCHUNK):
        slot = c % NBUF
        pltpu.make_async_copy(
            hid_ref.at[pl.ds(c * SUB, SUB), :],
            bufs[slot],
            sems[slot],
        ).wait()
        x = bufs[slot][...]
        lt = jax.lax.dot_general(
            wt, x, (((0,), (1,)), ((), ())),
            preferred_element_type=jnp.float32)
        if c + NBUF < NCHUNK:
            copy_in(c + NBUF)

        l8 = lt[:NUM_EXPERTS, :]

        m = jnp.max(l8, axis=0, keepdims=True)
        e = jnp.exp(l8 - m)
        s = jnp.sum(e, axis=0, keepdims=True)
        probs = e / s

        sel = jnp.zeros((1, SUB), jnp.int32)
        for k in range(NUM_EXPERTS - 1, -1, -1):
            sel = jnp.where(l8[k:k + 1, :] == m, k, sel)

        cols = pl.ds(c * SUB, SUB)
        logits_ref[:, cols] = l8
        sel_ref[:, cols] = sel
        wgt_ref[:, cols] = 1.0 / s

        ent_tok = -jnp.sum(probs * jnp.log(probs + 1e-8), axis=0,
                           keepdims=True)
        ent_acc[...] += jnp.sum(ent_tok).reshape(1, 1)
        load_acc[...] += jnp.sum(probs, axis=1, keepdims=True)

    load = load_acc[...] / NUM_TOKENS
    mean = jnp.sum(load) / NUM_EXPERTS
    var = jnp.sum((load - mean) ** 2) / NUM_EXPERTS
    var_ref[...] = var.reshape(1, 1)
    ent_ref[...] = ent_acc[...] / NUM_TOKENS


@jax.jit
def kernel(hidden_states, W):
    wt = jnp.pad(W.T, ((0, 0), (0, LANES - NUM_EXPERTS)))

    out_types = (
        jax.ShapeDtypeStruct((NUM_EXPERTS, NUM_TOKENS), jnp.float32),
        jax.ShapeDtypeStruct((1, NUM_TOKENS), jnp.int32),
        jax.ShapeDtypeStruct((1, NUM_TOKENS), jnp.float32),
        jax.ShapeDtypeStruct((1, 1), jnp.float32),
        jax.ShapeDtypeStruct((1, 1), jnp.float32),
    )
    logits_t, sel_t, wgt_t, var, ent = pl.pallas_call(
        _router_kernel,
        in_specs=[
            pl.BlockSpec(memory_space=pltpu.MemorySpace.HBM),
            pl.BlockSpec(memory_space=pltpu.MemorySpace.VMEM),
        ],
        out_specs=(
            pl.BlockSpec(memory_space=pltpu.MemorySpace.VMEM),
            pl.BlockSpec(memory_space=pltpu.MemorySpace.VMEM),
            pl.BlockSpec(memory_space=pltpu.MemorySpace.VMEM),
            pl.BlockSpec(memory_space=pltpu.MemorySpace.VMEM),
            pl.BlockSpec(memory_space=pltpu.MemorySpace.VMEM),
        ),
        out_shape=out_types,
        scratch_shapes=(
            [pltpu.VMEM((SUB, HIDDEN), jnp.float32) for _ in range(NBUF)]
            + [pltpu.VMEM((NUM_EXPERTS, 1), jnp.float32),
               pltpu.VMEM((1, 1), jnp.float32)]
            + [pltpu.SemaphoreType.DMA for _ in range(NBUF)]
        ),
    )(hidden_states, wt)

    return (logits_t.T, sel_t.reshape(NUM_TOKENS, 1),
            wgt_t.reshape(NUM_TOKENS, 1), var.reshape(()), ent.reshape(()))
